# baseline (speedup 1.0000x reference)
import functools

import jax
import jax.numpy as jnp
from jax import lax
from jax.experimental import pallas as pl
from jax.experimental.pallas import tpu as pltpu
from jax.experimental.pallas import tpu_sc as plsc

B = 16384
NF = 8
D = 64
HID = NF * D
VOCAB = 1000000
NITEMS = 1000000

NW = 32
RPW = B // NW
CH = 128
NCH = RPW // CH
NG = HID // 128


@functools.cache
def _mesh():
    return plsc.VectorSubcoreMesh(core_axis_name="c", subcore_axis_name="s")


@functools.cache
def _sc_gather():
    @functools.partial(
        pl.kernel,
        mesh=_mesh(),
        out_type=[
            jax.ShapeDtypeStruct((NF, B, 128), jnp.float32),
            jax.ShapeDtypeStruct((NF, B, 128), jnp.float32),
        ],
        scratch_types=[
            pltpu.VMEM((NF, RPW), jnp.int32),
            pltpu.VMEM((RPW,), jnp.int32),
            pltpu.VMEM((CH, 128), jnp.int32),
            pltpu.VMEM((NF, RPW), jnp.int32),
            pltpu.VMEM((2, CH, 128), jnp.float32),
            pltpu.SemaphoreType.DMA,
            pltpu.SemaphoreType.DMA,
        ],
        compiler_params=pltpu.CompilerParams(needs_layout_passes=False),
    )
    def body(ufT, item_ids, ift, tab, out_u, out_i,
             uidx, iid, frows, ifi, ebuf, gsem, wsem):
        w = lax.axis_index("s") * 2 + lax.axis_index("c")
        b0 = w * RPW

        pltpu.sync_copy(ufT.at[:, pl.ds(b0, RPW)], uidx)
        pltpu.sync_copy(item_ids.at[pl.ds(b0, RPW)], iid)

        for c in range(NCH):
            pltpu.async_copy(ift.at[iid.at[pl.ds(c * CH, CH)]], frows,
                             gsem).wait()
            for k in range(CH // 16):
                rows16 = lax.iota(jnp.int32, 16) + 16 * k
                for f in range(NF):
                    ifi[f, pl.ds(c * CH + 16 * k, 16)] = plsc.load_gather(
                        frows, [rows16, jnp.full((16,), f, jnp.int32)])

        for src, out in ((uidx, out_u), (ifi, out_i)):
            for f in range(NF):
                for c in range(NCH):
                    buf = ebuf.at[c % 2]
                    idx = src.at[f].at[pl.ds(c * CH, CH)]
                    pltpu.async_copy(tab.at[idx], buf, gsem).wait()
                    pltpu.sync_copy(
                        buf, out.at[f, pl.ds(b0 + c * CH, CH), :])

    return body


BLK = 1024
NB = B // BLK


def _mlp_body(ue, ie, uW1, ub1, uW2, ub2, uW3, ub3, iW1, ib1, iW2, ib2, out):
    f32 = jnp.float32

    def dot(a, b):
        return jnp.dot(a, b, preferred_element_type=f32)

    h = ub1[...]
    g = ib1[...]
    for f in range(NF):
        h = h + dot(ue[f][:, :D], uW1[pl.ds(D * f, D), :])
        g = g + dot(ie[f][:, :D], iW1[pl.ds(D * f, D), :])
    h = jnp.maximum(h, 0.0)
    g = jnp.maximum(g, 0.0)
    h = jnp.maximum(dot(h, uW2[...]) + ub2[...], 0.0)
    uv = dot(h, uW3[...]) + ub3[...]
    iv = dot(g, iW2[...]) + ib2[...]
    out[...] = jnp.sum(uv * iv, axis=1)


def _full(shape):
    return pl.BlockSpec(shape, lambda i: tuple(0 for _ in shape))


_mlp_call = pl.pallas_call(
    _mlp_body,
    grid=(NB,),
    in_specs=[
        pl.BlockSpec((NF, BLK, 128), lambda i: (0, i, 0)),
        pl.BlockSpec((NF, BLK, 128), lambda i: (0, i, 0)),
        _full((HID, HID // 2)),
        _full((1, HID // 2)),
        _full((HID // 2, HID // 4)),
        _full((1, HID // 4)),
        _full((HID // 4, D)),
        _full((1, D)),
        _full((HID, HID // 2)),
        _full((1, HID // 2)),
        _full((HID // 2, D)),
        _full((1, D)),
    ],
    out_specs=pl.BlockSpec((BLK,), lambda i: (i,)),
    out_shape=jax.ShapeDtypeStruct((B,), jnp.float32),
)


def kernel(user_feats, item_ids, item_feats_table, embed_table,
           uW1, ub1, uW2, ub2, uW3, ub3, iW1, ib1, iW2, ib2):
    tab128 = jnp.concatenate(
        [embed_table, jnp.zeros((VOCAB, 128 - D), jnp.float32)], axis=1)
    ift128 = jnp.concatenate(
        [item_feats_table, jnp.zeros((NITEMS, 128 - NF), jnp.int32)], axis=1)
    up, ip = _sc_gather()(user_feats.T, item_ids, ift128, tab128)
    scores = _mlp_call(up, ip,
                       uW1, ub1.reshape(1, -1), uW2, ub2.reshape(1, -1),
                       uW3, ub3.reshape(1, -1), iW1, ib1.reshape(1, -1),
                       iW2, ib2.reshape(1, -1))
    return scores


# speedup vs baseline: 1.1010x; 1.1010x over previous
import functools

import jax
import jax.numpy as jnp
from jax import lax
from jax.experimental import pallas as pl
from jax.experimental.pallas import tpu as pltpu
from jax.experimental.pallas import tpu_sc as plsc

B = 16384
NF = 8
D = 64
HID = NF * D
VOCAB = 1000000
NITEMS = 1000000

NW = 32
RPW = B // NW
CH = 128
NCH = RPW // CH
NG = HID // 128


@functools.cache
def _mesh():
    return plsc.VectorSubcoreMesh(core_axis_name="c", subcore_axis_name="s")


@functools.cache
def _sc_gather():
    @functools.partial(
        pl.kernel,
        mesh=_mesh(),
        out_type=[
            jax.ShapeDtypeStruct((NG, B, 128), jnp.float32),
            jax.ShapeDtypeStruct((NG, B, 128), jnp.float32),
        ],
        scratch_types=[
            pltpu.VMEM((NF, RPW), jnp.int32),
            pltpu.VMEM((RPW,), jnp.int32),
            pltpu.VMEM((CH, NF), jnp.int32),
            pltpu.VMEM((NF, RPW), jnp.int32),
            pltpu.VMEM((2, CH, D), jnp.float32),
            pltpu.SemaphoreType.DMA,
            pltpu.SemaphoreType.DMA,
        ],
        compiler_params=pltpu.CompilerParams(use_tc_tiling_on_sc=False,
                                             needs_layout_passes=False),
    )
    def body(ufT, item_ids, ift, tab, out_u, out_i,
             uidx, iid, frows, ifi, ebuf, gsem, wsem):
        w = lax.axis_index("s") * 2 + lax.axis_index("c")
        b0 = w * RPW

        pltpu.sync_copy(ufT.at[:, pl.ds(b0, RPW)], uidx)
        pltpu.sync_copy(item_ids.at[pl.ds(b0, RPW)], iid)

        for c in range(NCH):
            pltpu.async_copy(ift.at[iid.at[pl.ds(c * CH, CH)]], frows,
                             gsem).wait()
            for k in range(CH // 16):
                rows16 = lax.iota(jnp.int32, 16) + 16 * k
                for f in range(NF):
                    ifi[f, pl.ds(c * CH + 16 * k, 16)] = plsc.load_gather(
                        frows, [rows16, jnp.full((16,), f, jnp.int32)])

        items = [(src, out, f, c)
                 for src, out in ((uidx, out_u), (ifi, out_i))
                 for f in range(NF)
                 for c in range(NCH)]
        sems = (gsem, wsem)

        def fire(n):
            src, out, f, c = items[n]
            idx = src.at[f].at[pl.ds(c * CH, CH)]
            return pltpu.async_copy(tab.at[idx], ebuf.at[n % 2],
                                    sems[n % 2])

        hcur = fire(0)
        for n in range(len(items)):
            hnext = fire(n + 1) if n + 1 < len(items) else None
            hcur.wait()
            src, out, f, c = items[n]
            pltpu.sync_copy(
                ebuf.at[n % 2],
                out.at[f // 2, pl.ds(b0 + c * CH, CH), pl.ds((f % 2) * D, D)])
            hcur = hnext

    return body


BLK = 1024
NB = B // BLK


def _mlp_body(ue, ie, uW1, ub1, uW2, ub2, uW3, ub3, iW1, ib1, iW2, ib2, out):
    f32 = jnp.float32

    def dot(a, b):
        return jnp.dot(a, b, preferred_element_type=f32)

    h = ub1[...]
    g = ib1[...]
    for p in range(NG):
        h = h + dot(ue[p], uW1[pl.ds(128 * p, 128), :])
        g = g + dot(ie[p], iW1[pl.ds(128 * p, 128), :])
    h = jnp.maximum(h, 0.0)
    g = jnp.maximum(g, 0.0)
    h = jnp.maximum(dot(h, uW2[...]) + ub2[...], 0.0)
    uv = dot(h, uW3[...]) + ub3[...]
    iv = dot(g, iW2[...]) + ib2[...]
    out[...] = jnp.sum(uv * iv, axis=1)


def _full(shape):
    return pl.BlockSpec(shape, lambda i: tuple(0 for _ in shape))


_mlp_call = pl.pallas_call(
    _mlp_body,
    grid=(NB,),
    in_specs=[
        pl.BlockSpec((NG, BLK, 128), lambda i: (0, i, 0)),
        pl.BlockSpec((NG, BLK, 128), lambda i: (0, i, 0)),
        _full((HID, HID // 2)),
        _full((1, HID // 2)),
        _full((HID // 2, HID // 4)),
        _full((1, HID // 4)),
        _full((HID // 4, D)),
        _full((1, D)),
        _full((HID, HID // 2)),
        _full((1, HID // 2)),
        _full((HID // 2, D)),
        _full((1, D)),
    ],
    out_specs=pl.BlockSpec((BLK,), lambda i: (i,)),
    out_shape=jax.ShapeDtypeStruct((B,), jnp.float32),
)


def kernel(user_feats, item_ids, item_feats_table, embed_table,
           uW1, ub1, uW2, ub2, uW3, ub3, iW1, ib1, iW2, ib2):
    up, ip = _sc_gather()(user_feats.T, item_ids, item_feats_table,
                          embed_table)
    scores = _mlp_call(up, ip,
                       uW1, ub1.reshape(1, -1), uW2, ub2.reshape(1, -1),
                       uW3, ub3.reshape(1, -1), iW1, ib1.reshape(1, -1),
                       iW2, ib2.reshape(1, -1))
    return scores


# speedup vs baseline: 1.1105x; 1.0087x over previous
import functools

import jax
import jax.numpy as jnp
from jax import lax
from jax.experimental import pallas as pl
from jax.experimental.pallas import tpu as pltpu
from jax.experimental.pallas import tpu_sc as plsc

B = 16384
NF = 8
D = 64
HID = NF * D
VOCAB = 1000000
NITEMS = 1000000

NW = 32
RPW = B // NW
CH = 128
NCH = RPW // CH
NG = HID // 128


@functools.cache
def _mesh():
    return plsc.VectorSubcoreMesh(core_axis_name="c", subcore_axis_name="s")


@functools.cache
def _sc_gather():
    @functools.partial(
        pl.kernel,
        mesh=_mesh(),
        out_type=[
            jax.ShapeDtypeStruct((NG, B, 128), jnp.float32),
            jax.ShapeDtypeStruct((NG, B, 128), jnp.float32),
        ],
        scratch_types=[
            pltpu.VMEM((NF, RPW), jnp.int32),
            pltpu.VMEM((RPW,), jnp.int32),
            pltpu.VMEM((CH, NF), jnp.int32),
            pltpu.VMEM((NF, RPW), jnp.int32),
            pltpu.VMEM((4, CH, D), jnp.float32),
            pltpu.SemaphoreType.DMA,
            pltpu.SemaphoreType.DMA,
            pltpu.SemaphoreType.DMA,
            pltpu.SemaphoreType.DMA,
        ],
        compiler_params=pltpu.CompilerParams(use_tc_tiling_on_sc=False,
                                             needs_layout_passes=False),
    )
    def body(ufT, item_ids, ift, tab, out_u, out_i,
             uidx, iid, frows, ifi, ebuf, gsem, s1, s2, s3):
        w = lax.axis_index("s") * 2 + lax.axis_index("c")
        b0 = w * RPW

        pltpu.sync_copy(ufT.at[:, pl.ds(b0, RPW)], uidx)
        pltpu.sync_copy(item_ids.at[pl.ds(b0, RPW)], iid)

        for c in range(NCH):
            pltpu.async_copy(ift.at[iid.at[pl.ds(c * CH, CH)]], frows,
                             gsem).wait()
            for k in range(CH // 16):
                rows16 = lax.iota(jnp.int32, 16) + 16 * k
                for f in range(NF):
                    ifi[f, pl.ds(c * CH + 16 * k, 16)] = plsc.load_gather(
                        frows, [rows16, jnp.full((16,), f, jnp.int32)])

        items = [(src, out, f, c)
                 for src, out in ((uidx, out_u), (ifi, out_i))
                 for f in range(NF)
                 for c in range(NCH)]
        sems = (gsem, s1, s2, s3)
        DEPTH = 4

        def fire(n):
            src, out, f, c = items[n]
            idx = src.at[f].at[pl.ds(c * CH, CH)]
            return pltpu.async_copy(tab.at[idx], ebuf.at[n % DEPTH],
                                    sems[n % DEPTH])

        hs = [fire(n) for n in range(DEPTH - 1)]
        for n in range(len(items)):
            if n + DEPTH - 1 < len(items):
                hs.append(fire(n + DEPTH - 1))
            hs[n].wait()
            src, out, f, c = items[n]
            pltpu.sync_copy(
                ebuf.at[n % DEPTH],
                out.at[f // 2, pl.ds(b0 + c * CH, CH), pl.ds((f % 2) * D, D)])

    return body


BLK = 1024
NB = B // BLK


def _mlp_body(ue, ie, uW1, ub1, uW2, ub2, uW3, ub3, iW1, ib1, iW2, ib2, out):
    f32 = jnp.float32

    def dot(a, b):
        return jnp.dot(a, b, preferred_element_type=f32)

    h = ub1[...]
    g = ib1[...]
    for p in range(NG):
        h = h + dot(ue[p], uW1[pl.ds(128 * p, 128), :])
        g = g + dot(ie[p], iW1[pl.ds(128 * p, 128), :])
    h = jnp.maximum(h, 0.0)
    g = jnp.maximum(g, 0.0)
    h = jnp.maximum(dot(h, uW2[...]) + ub2[...], 0.0)
    uv = dot(h, uW3[...]) + ub3[...]
    iv = dot(g, iW2[...]) + ib2[...]
    out[...] = jnp.sum(uv * iv, axis=1)


def _full(shape):
    return pl.BlockSpec(shape, lambda i: tuple(0 for _ in shape))


_mlp_call = pl.pallas_call(
    _mlp_body,
    grid=(NB,),
    in_specs=[
        pl.BlockSpec((NG, BLK, 128), lambda i: (0, i, 0)),
        pl.BlockSpec((NG, BLK, 128), lambda i: (0, i, 0)),
        _full((HID, HID // 2)),
        _full((1, HID // 2)),
        _full((HID // 2, HID // 4)),
        _full((1, HID // 4)),
        _full((HID // 4, D)),
        _full((1, D)),
        _full((HID, HID // 2)),
        _full((1, HID // 2)),
        _full((HID // 2, D)),
        _full((1, D)),
    ],
    out_specs=pl.BlockSpec((BLK,), lambda i: (i,)),
    out_shape=jax.ShapeDtypeStruct((B,), jnp.float32),
)


def kernel(user_feats, item_ids, item_feats_table, embed_table,
           uW1, ub1, uW2, ub2, uW3, ub3, iW1, ib1, iW2, ib2):
    up, ip = _sc_gather()(user_feats.T, item_ids, item_feats_table,
                          embed_table)
    scores = _mlp_call(up, ip,
                       uW1, ub1.reshape(1, -1), uW2, ub2.reshape(1, -1),
                       uW3, ub3.reshape(1, -1), iW1, ib1.reshape(1, -1),
                       iW2, ib2.reshape(1, -1))
    return scores
